# baseline (speedup 1.0000x reference)
import jax
import jax.numpy as jnp
from jax.experimental import pallas as pl
from jax.experimental.pallas import tpu as pltpu

_N = 4096
_LO, _HI = 1024, 3072
_SPLIT = 4
_FILL_R = 512


def _body(b_ref, bt_ref, o_ref, fill_ref, sems):
    fill_ref[...] = jnp.full((_FILL_R, _HI - _LO), bt_ref[0, 0], jnp.float32)

    copies = []
    s = 0

    def _copy(src, dst):
        nonlocal s
        copies.append(pltpu.make_async_copy(src, dst, sems.at[s]))
        s += 1

    for base in (0, _HI):
        step = _LO // _SPLIT
        for i in range(_SPLIT):
            r = base + i * step
            _copy(b_ref.at[0, pl.ds(r, step), :], o_ref.at[0, pl.ds(r, step), :])

    step = (_HI - _LO) // 2
    for i in range(2):
        r = _LO + i * step
        _copy(b_ref.at[0, pl.ds(r, step), pl.ds(0, _LO)],
              o_ref.at[0, pl.ds(r, step), pl.ds(0, _LO)])
        _copy(b_ref.at[0, pl.ds(r, step), pl.ds(_HI, _N - _HI)],
              o_ref.at[0, pl.ds(r, step), pl.ds(_HI, _N - _HI)])

    for i in range((_HI - _LO) // _FILL_R):
        r = _LO + i * _FILL_R
        _copy(fill_ref.at[:, :], o_ref.at[0, pl.ds(r, _FILL_R), pl.ds(_LO, _HI - _LO)])

    for c in copies:
        c.start()
    for c in copies:
        c.wait()


def kernel(B, Bt):
    n_dma = 2 * _SPLIT + 4 + (_HI - _LO) // _FILL_R
    return pl.pallas_call(
        _body,
        in_specs=[
            pl.BlockSpec(memory_space=pl.ANY),
            pl.BlockSpec(memory_space=pltpu.SMEM),
        ],
        out_specs=pl.BlockSpec(memory_space=pl.ANY),
        out_shape=jax.ShapeDtypeStruct((1, _N, _N), jnp.float32),
        scratch_shapes=[
            pltpu.VMEM((_FILL_R, _HI - _LO), jnp.float32),
            pltpu.SemaphoreType.DMA((n_dma,)),
        ],
    )(B, Bt)


# speedup vs baseline: 37.9197x; 37.9197x over previous
import jax
import jax.numpy as jnp
from jax.experimental import pallas as pl
from jax.experimental.pallas import tpu as pltpu

_N = 4096
_LO, _HI = 1024, 3072
_BR = 256
_M0, _M1 = _LO // _BR, _HI // _BR


def _body(full_ref, left_ref, right_ref, bt_ref, o_ref):
    i = pl.program_id(0)
    in_rows = (i >= _M0) & (i < _M1)

    @pl.when(in_rows)
    def _mid():
        o_ref[:, :, : _LO] = left_ref[...]
        o_ref[:, :, _LO:_HI] = jnp.full((1, _BR, _HI - _LO), bt_ref[0, 0],
                                        jnp.float32)
        o_ref[:, :, _HI:] = right_ref[...]

    @pl.when(jnp.logical_not(in_rows))
    def _copy():
        o_ref[...] = full_ref[...]


def _full_idx(i):
    return (0, jnp.where((i >= _M0) & (i < _M1), _M0 - 1, i), 0)


def _slab_idx(col_block):
    def idx(i):
        return (0, jnp.clip(i, _M0, _M1 - 1), col_block)
    return idx


def kernel(B, Bt):
    return pl.pallas_call(
        _body,
        grid=(_N // _BR,),
        in_specs=[
            pl.BlockSpec((1, _BR, _N), _full_idx),
            pl.BlockSpec((1, _BR, _LO), _slab_idx(0)),
            pl.BlockSpec((1, _BR, _N - _HI), _slab_idx(_HI // (_N - _HI))),
            pl.BlockSpec(memory_space=pltpu.SMEM),
        ],
        out_specs=pl.BlockSpec((1, _BR, _N), lambda i: (0, i, 0)),
        out_shape=jax.ShapeDtypeStruct((1, _N, _N), jnp.float32),
    )(B, B, B, Bt)


# speedup vs baseline: 40.2529x; 1.0615x over previous
import jax
import jax.numpy as jnp
from jax.experimental import pallas as pl
from jax.experimental.pallas import tpu as pltpu

_N = 4096
_LO, _HI = 1024, 3072
_BR = 512
_M0, _M1 = _LO // _BR, _HI // _BR


def _body(full_ref, left_ref, right_ref, bt_ref, o_ref):
    i = pl.program_id(0)
    in_rows = (i >= _M0) & (i < _M1)

    @pl.when(in_rows)
    def _mid():
        o_ref[:, :, : _LO] = left_ref[...]
        o_ref[:, :, _LO:_HI] = jnp.full((1, _BR, _HI - _LO), bt_ref[0, 0],
                                        jnp.float32)
        o_ref[:, :, _HI:] = right_ref[...]

    @pl.when(jnp.logical_not(in_rows))
    def _copy():
        o_ref[...] = full_ref[...]


def _full_idx(i):
    return (0, jnp.where((i >= _M0) & (i < _M1), _M0 - 1, i), 0)


def _slab_idx(col_block):
    def idx(i):
        return (0, jnp.clip(i, _M0, _M1 - 1), col_block)
    return idx


def kernel(B, Bt):
    return pl.pallas_call(
        _body,
        grid=(_N // _BR,),
        in_specs=[
            pl.BlockSpec((1, _BR, _N), _full_idx),
            pl.BlockSpec((1, _BR, _LO), _slab_idx(0)),
            pl.BlockSpec((1, _BR, _N - _HI), _slab_idx(_HI // (_N - _HI))),
            pl.BlockSpec(memory_space=pltpu.SMEM),
        ],
        out_specs=pl.BlockSpec((1, _BR, _N), lambda i: (0, i, 0)),
        out_shape=jax.ShapeDtypeStruct((1, _N, _N), jnp.float32),
    )(B, B, B, Bt)
